# Initial kernel scaffold; baseline (speedup 1.0000x reference)
import jax
import jax.numpy as jnp
from jax.experimental import pallas as pl


def kernel(src_features, edge_sh, edge_emb, src, dst, mlp_w0, mlp_b0, mlp_w1, mlp_b1):
    raise NotImplementedError("write your pallas kernel here")


# speedup vs baseline: 1.7127x; 1.7127x over previous
import functools

import numpy as np
import jax
import jax.numpy as jnp
from jax import lax
from jax.experimental import pallas as pl
from jax.experimental.pallas import tpu as pltpu

E = 80000
N = 10000
MUL = 16

_PW_S = np.float32(1.0 / np.sqrt(32.0))
_PW_VI = np.float32(np.sqrt(3.0 / 32.0) / np.sqrt(3.0))
_INV3 = np.float32(1.0 / np.sqrt(3.0))
_ISQ2 = np.float32(1.0 / np.sqrt(2.0))

_IT = False


def _consts():
    r256 = np.kron(np.eye(16), np.ones((1, 16))).astype(np.float32)
    s256 = np.kron(np.ones((16, 1)), np.eye(16)).astype(np.float32)
    rv = np.kron(np.ones((1, 16)), np.eye(3)).astype(np.float32)
    t3 = np.kron(np.eye(16), np.ones((3, 1))).astype(np.float32)
    ec = np.zeros((48, 48), np.float32)
    for u in range(16):
        for j in range(3):
            ec[3 * u + j, 16 * j + u] = 1.0
    ea = np.kron(np.eye(16), np.ones((1, 3))).astype(np.float32)
    eb = np.zeros((48, 48), np.float32)
    for k in range(3):
        for w in range(16):
            eb[16 * k + w, 3 * w + k] = 1.0
    return r256, s256, rv, t3, ec, ea, eb


def _tp_body(x1_ref, sh_ref, emb_ref, w0t_ref, b0_ref, w1t_ref, b1_ref,
             r_ref, s_ref, rv_ref, t3_ref, ec_ref, ea_ref, eb_ref, out_ref):
    h = emb_ref[...] @ w0t_ref[...] + b0_ref[...]
    h = 0.5 * h * (1.0 + lax.erf(h * _ISQ2))
    tpw = h @ w1t_ref[...] + b1_ref[...]

    x1 = x1_ref[...]
    sh = sh_ref[...]
    s1 = x1[:, :16]
    x1v = x1[:, 16:]
    s2 = sh[:, 0:1]
    v2 = sh[:, 1:4]

    v2rep = v2 @ rv_ref[...]
    dot12 = (x1v * v2rep) @ t3_ref[...]
    v3 = x1v @ ec_ref[...]

    r = r_ref[...]
    s = s_ref[...]

    def path(a, wblk):
        return ((a @ r) * wblk) @ s

    w000 = tpw[:, 0:256]
    w011 = tpw[:, 256:512]
    w101 = tpw[:, 512:768]
    w110 = tpw[:, 768:1024]

    e000 = path(s1 * s2, w000)
    e110 = path(dot12, w110)
    e011 = path(s1, w011)
    f101 = jnp.concatenate(
        [path(v3[:, 0:16], w101),
         path(v3[:, 16:32], w101),
         path(v3[:, 32:48], w101)], axis=1)

    out_s = _PW_S * (e000 + _INV3 * e110)
    term1 = (e011 @ ea_ref[...]) * v2rep
    term2 = (f101 @ eb_ref[...]) * s2
    out_v = _PW_VI * (term1 + term2)
    out_ref[...] = jnp.concatenate([out_s, out_v], axis=1)


def _tc_main(x1, edge_sh, edge_emb, w0, b0, w1, b1):
    bsz = 800
    consts = [jnp.asarray(c) for c in _consts()]

    def dspec(cols):
        return pl.BlockSpec((bsz, cols), lambda i: (i, 0))

    def fspec(shape):
        return pl.BlockSpec(shape, lambda i: (0,) * len(shape))

    return pl.pallas_call(
        _tp_body,
        grid=(E // bsz,),
        in_specs=[dspec(64), dspec(4), dspec(64),
                  fspec((64, 64)), fspec((1, 64)),
                  fspec((64, 1024)), fspec((1, 1024)),
                  fspec((16, 256)), fspec((256, 16)), fspec((3, 48)),
                  fspec((48, 16)), fspec((48, 48)), fspec((16, 48)),
                  fspec((48, 48))],
        out_specs=dspec(64),
        out_shape=jax.ShapeDtypeStruct((E, 64), jnp.float32),
        interpret=_IT,
    )(x1, edge_sh, edge_emb, w0.T, b0.reshape(1, 64), w1.T,
      b1.reshape(1, 1024), *consts)


def _combine_body(a0_ref, a1_ref, c0_ref, c1_ref, out_ref):
    cnt = c0_ref[:, 0:1] + c1_ref[:, 0:1]
    out_ref[...] = (a0_ref[...] + a1_ref[...]) / jnp.maximum(cnt, 1.0)


def _tc_combine(acc, cnt):
    return pl.pallas_call(
        _combine_body,
        out_shape=jax.ShapeDtypeStruct((N, 64), jnp.float32),
        interpret=_IT,
    )(acc[0], acc[1], cnt[0], cnt[1])


def kernel(src_features, edge_sh, edge_emb, src, dst,
           mlp_w0, mlp_b0, mlp_w1, mlp_b1):
    x1 = src_features[src]
    out_e = _tc_main(x1, edge_sh, edge_emb, mlp_w0, mlp_b0, mlp_w1, mlp_b1)
    acc = jax.ops.segment_sum(out_e, dst, num_segments=N)
    cnt = jax.ops.segment_sum(jnp.ones((E,), jnp.float32), dst, num_segments=N)
    return acc / jnp.maximum(cnt, 1.0)[:, None]


# speedup vs baseline: 3.5220x; 2.0564x over previous
import functools

import numpy as np
import jax
import jax.numpy as jnp
from jax import lax
from jax.experimental import pallas as pl
from jax.experimental.pallas import tpu as pltpu
from jax.experimental.pallas import tpu_sc as plsc

E = 80000
N = 10000
MUL = 16

_PW_S = np.float32(1.0 / np.sqrt(32.0))
_PW_VI = np.float32(np.sqrt(3.0 / 32.0) / np.sqrt(3.0))
_INV3 = np.float32(1.0 / np.sqrt(3.0))
_ISQ2 = np.float32(1.0 / np.sqrt(2.0))

_IT = False


def _consts():
    r256 = np.kron(np.eye(16), np.ones((1, 16))).astype(np.float32)
    s256 = np.kron(np.ones((16, 1)), np.eye(16)).astype(np.float32)
    rv = np.kron(np.ones((1, 16)), np.eye(3)).astype(np.float32)
    t3 = np.kron(np.eye(16), np.ones((3, 1))).astype(np.float32)
    ec = np.zeros((48, 48), np.float32)
    for u in range(16):
        for j in range(3):
            ec[3 * u + j, 16 * j + u] = 1.0
    ea = np.kron(np.eye(16), np.ones((1, 3))).astype(np.float32)
    eb = np.zeros((48, 48), np.float32)
    for k in range(3):
        for w in range(16):
            eb[16 * k + w, 3 * w + k] = 1.0
    return r256, s256, rv, t3, ec, ea, eb


def _tp_body(x1_ref, sh_ref, emb_ref, w0t_ref, b0_ref, w1t_ref, b1_ref,
             r_ref, s_ref, rv_ref, t3_ref, ec_ref, ea_ref, eb_ref, out_ref):
    bsz = out_ref.shape[0]
    h = emb_ref[...] @ w0t_ref[...] + b0_ref[...]
    h = 0.5 * h * (1.0 + lax.erf(h * _ISQ2))
    tpw = h @ w1t_ref[...] + b1_ref[...]

    x1 = x1_ref[...]
    sh = sh_ref[...]
    s1 = x1[:, :16]
    x1v = x1[:, 16:64]
    s2 = sh[:, 0:1]
    v2 = sh[:, 1:4]

    v2rep = v2 @ rv_ref[...]
    dot12 = (x1v * v2rep) @ t3_ref[...]
    v3 = x1v @ ec_ref[...]

    r = r_ref[...]
    s = s_ref[...]

    def path(a, wblk):
        return ((a @ r) * wblk) @ s

    w000 = tpw[:, 0:256]
    w011 = tpw[:, 256:512]
    w101 = tpw[:, 512:768]
    w110 = tpw[:, 768:1024]

    e000 = path(s1 * s2, w000)
    e110 = path(dot12, w110)
    e011 = path(s1, w011)
    f101 = jnp.concatenate(
        [path(v3[:, 0:16], w101),
         path(v3[:, 16:32], w101),
         path(v3[:, 32:48], w101)], axis=1)

    out_s = _PW_S * (e000 + _INV3 * e110)
    term1 = (e011 @ ea_ref[...]) * v2rep
    term2 = (f101 @ eb_ref[...]) * s2
    out_v = _PW_VI * (term1 + term2)
    pad = jnp.zeros((bsz, 63), jnp.float32)
    one = jnp.ones((bsz, 1), jnp.float32)
    out_ref[...] = jnp.concatenate([out_s, out_v, one, pad], axis=1)


def _tc_main(x1, edge_sh, edge_emb, w0, b0, w1, b1):
    bsz = 800
    consts = [jnp.asarray(c) for c in _consts()]

    def dspec(cols):
        return pl.BlockSpec((bsz, cols), lambda i: (i, 0))

    def fspec(shape):
        return pl.BlockSpec(shape, lambda i: (0,) * len(shape))

    return pl.pallas_call(
        _tp_body,
        grid=(E // bsz,),
        in_specs=[dspec(128), dspec(4), dspec(64),
                  fspec((64, 64)), fspec((1, 64)),
                  fspec((64, 1024)), fspec((1, 1024)),
                  fspec((16, 256)), fspec((256, 16)), fspec((3, 48)),
                  fspec((48, 16)), fspec((48, 48)), fspec((16, 48)),
                  fspec((48, 48))],
        out_specs=dspec(128),
        out_shape=jax.ShapeDtypeStruct((E, 128), jnp.float32),
        interpret=_IT,
    )(x1, edge_sh, edge_emb, w0.T, b0.reshape(1, 64), w1.T,
      b1.reshape(1, 1024), *consts)


def _combine_body(a0_ref, a1_ref, out_ref):
    a = a0_ref[...] + a1_ref[...]
    cnt = a[:, 64:65]
    out_ref[...] = a[:, :64] / jnp.maximum(cnt, 1.0)


def _tc_combine(acc):
    return pl.pallas_call(
        _combine_body,
        out_shape=jax.ShapeDtypeStruct((N, 64), jnp.float32),
        interpret=_IT,
    )(acc[0], acc[1])


_CH = 128
_NCH = E // _CH
_NW = 32
_BASE_CH = _NCH // _NW
_EXTRA = _NCH % _NW
_RPT = 632
_RPT_TAIL = N - 15 * _RPT


def _sc_gather(table, idx):
    mesh = plsc.VectorSubcoreMesh(core_axis_name="c", subcore_axis_name="s")

    @functools.partial(
        pl.kernel, mesh=mesh,
        out_type=jax.ShapeDtypeStruct((E, 128), jnp.float32),
        scratch_types=[pltpu.VMEM((_CH,), jnp.int32),
                       pltpu.VMEM((_CH, 128), jnp.float32),
                       pltpu.SemaphoreType.DMA],
    )
    def gk(table_hbm, idx_hbm, out_hbm, idx_v, rows_v, sem):
        wid = lax.axis_index("s") * 2 + lax.axis_index("c")
        nch = _BASE_CH + jnp.where(wid < _EXTRA, 1, 0)

        def body(i, carry):
            base = pl.multiple_of((wid + i * _NW) * _CH, _CH)
            pltpu.sync_copy(idx_hbm.at[pl.ds(base, _CH)], idx_v)
            pltpu.async_copy(table_hbm.at[idx_v], rows_v, sem).wait()
            pltpu.sync_copy(rows_v, out_hbm.at[pl.ds(base, _CH)])
            return carry

        lax.fori_loop(0, nch, body, 0)

    return gk(table, idx)


def _sc_scatter(out_e, dst, z128):
    mesh = plsc.VectorSubcoreMesh(core_axis_name="c", subcore_axis_name="s")

    @functools.partial(
        pl.kernel, mesh=mesh,
        out_type=jax.ShapeDtypeStruct((2, N, 128), jnp.float32),
        scratch_types=[pltpu.VMEM((_CH,), jnp.int32),
                       pltpu.VMEM((_CH, 128), jnp.float32),
                       pltpu.VMEM_SHARED((N, 128), jnp.float32),
                       pltpu.SemaphoreType.DMA],
    )
    def sk(oute_hbm, dst_hbm, z_hbm, acc_hbm, idx_v, rows_v, acc_sh, sem):
        cid = lax.axis_index("c")
        sid = lax.axis_index("s")
        wid = sid * 2 + cid
        r0 = pl.multiple_of(sid * _RPT, 8)

        @pl.when(sid < 15)
        def _():
            pltpu.sync_copy(z_hbm.at[pl.ds(r0, _RPT)],
                            acc_sh.at[pl.ds(r0, _RPT)])

        @pl.when(sid == 15)
        def _():
            pltpu.sync_copy(z_hbm.at[pl.ds(15 * _RPT, _RPT_TAIL)],
                            acc_sh.at[pl.ds(15 * _RPT, _RPT_TAIL)])

        plsc.subcore_barrier()

        nch = _BASE_CH + jnp.where(wid < _EXTRA, 1, 0)

        def body(i, carry):
            base = pl.multiple_of((wid + i * _NW) * _CH, _CH)
            pltpu.sync_copy(dst_hbm.at[pl.ds(base, _CH)], idx_v)
            pltpu.sync_copy(oute_hbm.at[pl.ds(base, _CH)], rows_v)
            pltpu.sync_copy(rows_v, acc_sh.at[idx_v], add=True)
            return carry

        lax.fori_loop(0, nch, body, 0)
        plsc.subcore_barrier()

        @pl.when(sid < 15)
        def _():
            pltpu.sync_copy(acc_sh.at[pl.ds(r0, _RPT)],
                            acc_hbm.at[cid, pl.ds(r0, _RPT)])

        @pl.when(sid == 15)
        def _():
            pltpu.sync_copy(acc_sh.at[pl.ds(15 * _RPT, _RPT_TAIL)],
                            acc_hbm.at[cid, pl.ds(15 * _RPT, _RPT_TAIL)])

    return sk(out_e, dst, z128)


def kernel(src_features, edge_sh, edge_emb, src, dst,
           mlp_w0, mlp_b0, mlp_w1, mlp_b1):
    table = jnp.pad(src_features, ((0, 0), (0, 64)))
    x1 = _sc_gather(table, src)
    out_e = _tc_main(x1, edge_sh, edge_emb, mlp_w0, mlp_b0, mlp_w1, mlp_b1)
    z128 = jnp.zeros((N, 128), jnp.float32)
    acc = _sc_scatter(out_e, dst, z128)
    return _tc_combine(acc)


# speedup vs baseline: 3.9374x; 1.1179x over previous
import functools

import numpy as np
import jax
import jax.numpy as jnp
from jax import lax
from jax.experimental import pallas as pl
from jax.experimental.pallas import tpu as pltpu
from jax.experimental.pallas import tpu_sc as plsc

E = 80000
N = 10000
MUL = 16

_PW_S = np.float32(1.0 / np.sqrt(32.0))
_PW_VI = np.float32(np.sqrt(3.0 / 32.0) / np.sqrt(3.0))
_INV3 = np.float32(1.0 / np.sqrt(3.0))
_ISQ2 = np.float32(1.0 / np.sqrt(2.0))

_IT = False


def _consts():
    r = np.kron(np.eye(16), np.ones((1, 16))).astype(np.float32)
    s = (np.kron(np.ones((16, 1)), np.eye(16)) * _PW_S).astype(np.float32)
    rv = np.kron(np.ones((1, 16)), np.eye(3)).astype(np.float32)
    t3r = np.zeros((48, 256), np.float32)
    for u in range(16):
        for k in range(3):
            for w in range(16):
                t3r[3 * u + k, 16 * u + w] = _INV3
    ecr = np.zeros((48, 768), np.float32)
    for u in range(16):
        for j in range(3):
            for w in range(16):
                ecr[3 * u + j, 256 * j + 16 * u + w] = 1.0
    sea = np.zeros((256, 48), np.float32)
    for u in range(16):
        for w in range(16):
            for k in range(3):
                sea[16 * u + w, 3 * w + k] = _PW_VI
    seb = np.zeros((768, 48), np.float32)
    for k in range(3):
        for u in range(16):
            for w in range(16):
                seb[256 * k + 16 * u + w, 3 * w + k] = _PW_VI
    return r, s, rv, t3r, ecr, sea, seb


def _tp_body(x1_ref, sh_ref, emb_ref, w0t_ref, b0_ref, w1t_ref, b1_ref,
             r_ref, s_ref, rv_ref, t3r_ref, ecr_ref, sea_ref, seb_ref,
             out_ref):
    bsz = out_ref.shape[0]
    h = emb_ref[...] @ w0t_ref[...] + b0_ref[...]
    h = 0.5 * h * (1.0 + lax.erf(h * _ISQ2))
    tpw = jnp.dot(h.astype(jnp.bfloat16), w1t_ref[...],
                  preferred_element_type=jnp.float32) + b1_ref[...]

    x1 = x1_ref[...]
    sh = sh_ref[...]
    s1 = x1[:, :16]
    x1v = x1[:, 16:64]
    s2 = sh[:, 0:1]
    v2 = sh[:, 1:4]

    v2rep = v2 @ rv_ref[...]
    a011 = s1 @ r_ref[...]
    a000 = a011 * s2
    a110 = (x1v * v2rep) @ t3r_ref[...]
    a101 = x1v @ ecr_ref[...]

    w000 = tpw[:, 0:256]
    w011 = tpw[:, 256:512]
    w101 = tpw[:, 512:768]
    w110 = tpw[:, 768:1024]
    w101x3 = jnp.concatenate([w101, w101, w101], axis=1)

    ps = a000 * w000 + a110 * w110
    out_s = ps @ s_ref[...]
    term1 = ((a011 * w011) @ sea_ref[...]) * v2rep
    term2 = ((a101 * w101x3) @ seb_ref[...]) * s2
    out_v = term1 + term2
    pad = jnp.zeros((bsz, 63), jnp.float32)
    one = jnp.ones((bsz, 1), jnp.float32)
    out_ref[...] = jnp.concatenate([out_s, out_v, one, pad], axis=1)


def _tc_main(x1, edge_sh, edge_emb, w0, b0, w1, b1):
    bsz = 800
    consts = [jnp.asarray(c) for c in _consts()]

    def dspec(cols):
        return pl.BlockSpec((bsz, cols), lambda i: (i, 0))

    def fspec(shape):
        return pl.BlockSpec(shape, lambda i: (0,) * len(shape))

    return pl.pallas_call(
        _tp_body,
        grid=(E // bsz,),
        in_specs=[dspec(128), dspec(4), dspec(64),
                  fspec((64, 64)), fspec((1, 64)),
                  fspec((64, 1024)), fspec((1, 1024)),
                  fspec((16, 256)), fspec((256, 16)), fspec((3, 48)),
                  fspec((48, 256)), fspec((48, 768)), fspec((256, 48)),
                  fspec((768, 48))],
        out_specs=dspec(128),
        out_shape=jax.ShapeDtypeStruct((E, 128), jnp.float32),
        interpret=_IT,
    )(x1, edge_sh, edge_emb, w0.T, b0.reshape(1, 64),
      w1.T.astype(jnp.bfloat16), b1.reshape(1, 1024), *consts)


def _combine_body(a0_ref, a1_ref, out_ref):
    a = a0_ref[...] + a1_ref[...]
    cnt = a[:, 64:65]
    out_ref[...] = a[:, :64] / jnp.maximum(cnt, 1.0)


def _tc_combine(acc):
    return pl.pallas_call(
        _combine_body,
        out_shape=jax.ShapeDtypeStruct((N, 64), jnp.float32),
        interpret=_IT,
    )(acc[0], acc[1])


_CH = 128
_NCH = E // _CH
_NW = 32
_BASE_CH = _NCH // _NW
_EXTRA = _NCH % _NW
_RPT = 632
_RPT_TAIL = N - 15 * _RPT


def _sc_gather(table, idx):
    mesh = plsc.VectorSubcoreMesh(core_axis_name="c", subcore_axis_name="s")

    @functools.partial(
        pl.kernel, mesh=mesh,
        out_type=jax.ShapeDtypeStruct((E, 128), jnp.float32),
        scratch_types=[pltpu.VMEM((_CH,), jnp.int32),
                       pltpu.VMEM((_CH, 128), jnp.float32),
                       pltpu.SemaphoreType.DMA],
    )
    def gk(table_hbm, idx_hbm, out_hbm, idx_v, rows_v, sem):
        wid = lax.axis_index("s") * 2 + lax.axis_index("c")
        nch = _BASE_CH + jnp.where(wid < _EXTRA, 1, 0)

        def body(i, carry):
            base = pl.multiple_of((wid + i * _NW) * _CH, _CH)
            pltpu.sync_copy(idx_hbm.at[pl.ds(base, _CH)], idx_v)
            pltpu.async_copy(table_hbm.at[idx_v], rows_v, sem).wait()
            pltpu.sync_copy(rows_v, out_hbm.at[pl.ds(base, _CH)])
            return carry

        lax.fori_loop(0, nch, body, 0)

    return gk(table, idx)


def _sc_scatter(out_e, dst, z128):
    mesh = plsc.VectorSubcoreMesh(core_axis_name="c", subcore_axis_name="s")

    @functools.partial(
        pl.kernel, mesh=mesh,
        out_type=jax.ShapeDtypeStruct((2, N, 128), jnp.float32),
        scratch_types=[pltpu.VMEM((_CH,), jnp.int32),
                       pltpu.VMEM((_CH, 128), jnp.float32),
                       pltpu.VMEM_SHARED((N, 128), jnp.float32),
                       pltpu.SemaphoreType.DMA],
    )
    def sk(oute_hbm, dst_hbm, z_hbm, acc_hbm, idx_v, rows_v, acc_sh, sem):
        cid = lax.axis_index("c")
        sid = lax.axis_index("s")
        wid = sid * 2 + cid
        r0 = pl.multiple_of(sid * _RPT, 8)

        @pl.when(sid < 15)
        def _():
            pltpu.sync_copy(z_hbm.at[pl.ds(r0, _RPT)],
                            acc_sh.at[pl.ds(r0, _RPT)])

        @pl.when(sid == 15)
        def _():
            pltpu.sync_copy(z_hbm.at[pl.ds(15 * _RPT, _RPT_TAIL)],
                            acc_sh.at[pl.ds(15 * _RPT, _RPT_TAIL)])

        plsc.subcore_barrier()

        nch = _BASE_CH + jnp.where(wid < _EXTRA, 1, 0)

        def body(i, carry):
            base = pl.multiple_of((wid + i * _NW) * _CH, _CH)
            pltpu.sync_copy(dst_hbm.at[pl.ds(base, _CH)], idx_v)
            pltpu.sync_copy(oute_hbm.at[pl.ds(base, _CH)], rows_v)
            pltpu.sync_copy(rows_v, acc_sh.at[idx_v], add=True)
            return carry

        lax.fori_loop(0, nch, body, 0)
        plsc.subcore_barrier()

        @pl.when(sid < 15)
        def _():
            pltpu.sync_copy(acc_sh.at[pl.ds(r0, _RPT)],
                            acc_hbm.at[cid, pl.ds(r0, _RPT)])

        @pl.when(sid == 15)
        def _():
            pltpu.sync_copy(acc_sh.at[pl.ds(15 * _RPT, _RPT_TAIL)],
                            acc_hbm.at[cid, pl.ds(15 * _RPT, _RPT_TAIL)])

    return sk(out_e, dst, z128)


def kernel(src_features, edge_sh, edge_emb, src, dst,
           mlp_w0, mlp_b0, mlp_w1, mlp_b1):
    table = jnp.pad(src_features, ((0, 0), (0, 64)))
    x1 = _sc_gather(table, src)
    out_e = _tc_main(x1, edge_sh, edge_emb, mlp_w0, mlp_b0, mlp_w1, mlp_b1)
    z128 = jnp.zeros((N, 128), jnp.float32)
    acc = _sc_scatter(out_e, dst, z128)
    return _tc_combine(acc)


# speedup vs baseline: 4.1716x; 1.0595x over previous
import functools

import numpy as np
import jax
import jax.numpy as jnp
from jax import lax
from jax.experimental import pallas as pl
from jax.experimental.pallas import tpu as pltpu
from jax.experimental.pallas import tpu_sc as plsc

E = 80000
N = 10000
MUL = 16

_PW_S = np.float32(1.0 / np.sqrt(32.0))
_PW_VI = np.float32(np.sqrt(3.0 / 32.0) / np.sqrt(3.0))
_INV3 = np.float32(1.0 / np.sqrt(3.0))
_ISQ2 = np.float32(1.0 / np.sqrt(2.0))

_IT = False


def _consts():
    r = np.kron(np.eye(16), np.ones((1, 16))).astype(np.float32)
    s = (np.kron(np.ones((16, 1)), np.eye(16)) * _PW_S).astype(np.float32)
    rv = np.kron(np.ones((1, 16)), np.eye(3)).astype(np.float32)
    t3r = np.zeros((48, 256), np.float32)
    for u in range(16):
        for k in range(3):
            for w in range(16):
                t3r[3 * u + k, 16 * u + w] = _INV3
    ecr = np.zeros((48, 768), np.float32)
    for u in range(16):
        for j in range(3):
            for w in range(16):
                ecr[3 * u + j, 256 * j + 16 * u + w] = 1.0
    sea = np.zeros((256, 48), np.float32)
    for u in range(16):
        for w in range(16):
            for k in range(3):
                sea[16 * u + w, 3 * w + k] = _PW_VI
    seb = np.zeros((768, 48), np.float32)
    for k in range(3):
        for u in range(16):
            for w in range(16):
                seb[256 * k + 16 * u + w, 3 * w + k] = _PW_VI
    return r, s, rv, t3r, ecr, sea, seb


def _tp_body(x1_ref, sh_ref, emb_ref, w0t_ref, b0_ref, w1t_ref, b1_ref,
             r_ref, s_ref, rv_ref, t3r_ref, ecr_ref, sea_ref, seb_ref,
             out_ref):
    bsz = out_ref.shape[0]
    h = emb_ref[...] @ w0t_ref[...] + b0_ref[...]
    h = 0.5 * h * (1.0 + lax.erf(h * _ISQ2))
    tpw = jnp.dot(h.astype(jnp.bfloat16), w1t_ref[...],
                  preferred_element_type=jnp.float32) + b1_ref[...]

    x1 = x1_ref[...]
    sh = sh_ref[...]
    s1 = x1[:, :16]
    x1v = x1[:, 16:64]
    s2 = sh[:, 0:1]
    v2 = sh[:, 1:4]

    v2rep = v2 @ rv_ref[...]
    a011 = s1 @ r_ref[...]
    a000 = a011 * s2
    a110 = (x1v * v2rep) @ t3r_ref[...]
    a101 = x1v @ ecr_ref[...]

    w000 = tpw[:, 0:256]
    w011 = tpw[:, 256:512]
    w101 = tpw[:, 512:768]
    w110 = tpw[:, 768:1024]
    w101x3 = jnp.concatenate([w101, w101, w101], axis=1)

    ps = a000 * w000 + a110 * w110
    out_s = ps @ s_ref[...]
    term1 = ((a011 * w011) @ sea_ref[...]) * v2rep
    term2 = ((a101 * w101x3) @ seb_ref[...]) * s2
    out_v = term1 + term2
    pad = jnp.zeros((bsz, 63), jnp.float32)
    one = jnp.ones((bsz, 1), jnp.float32)
    out_ref[...] = jnp.concatenate([out_s, out_v, one, pad], axis=1)


def _tc_main(x1, edge_sh, edge_emb, w0, b0, w1, b1):
    bsz = 800
    consts = [jnp.asarray(c) for c in _consts()]

    def dspec(cols):
        return pl.BlockSpec((bsz, cols), lambda i: (i, 0))

    def fspec(shape):
        return pl.BlockSpec(shape, lambda i: (0,) * len(shape))

    return pl.pallas_call(
        _tp_body,
        grid=(E // bsz,),
        in_specs=[dspec(128), dspec(4), dspec(64),
                  fspec((64, 64)), fspec((1, 64)),
                  fspec((64, 1024)), fspec((1, 1024)),
                  fspec((16, 256)), fspec((256, 16)), fspec((3, 48)),
                  fspec((48, 256)), fspec((48, 768)), fspec((256, 48)),
                  fspec((768, 48))],
        out_specs=dspec(128),
        out_shape=jax.ShapeDtypeStruct((E, 128), jnp.float32),
        interpret=_IT,
    )(x1, edge_sh, edge_emb, w0.T, b0.reshape(1, 64),
      w1.T.astype(jnp.bfloat16), b1.reshape(1, 1024), *consts)


def _combine_body(a0_ref, a1_ref, out_ref):
    a = a0_ref[...] + a1_ref[...]
    cnt = a[:, 64:65]
    out_ref[...] = a[:, :64] / jnp.maximum(cnt, 1.0)


def _tc_combine(acc):
    return pl.pallas_call(
        _combine_body,
        out_shape=jax.ShapeDtypeStruct((N, 64), jnp.float32),
        interpret=_IT,
    )(acc[0], acc[1])


_CH = 128
_NCH = E // _CH
_NW = 32
_BASE_CH = _NCH // _NW
_EXTRA = _NCH % _NW
_RPT = 632
_RPT_TAIL = N - 15 * _RPT


_NSLOT = _BASE_CH + 1


def _sc_gather(table, idx):
    mesh = plsc.VectorSubcoreMesh(core_axis_name="c", subcore_axis_name="s")

    @functools.partial(
        pl.kernel, mesh=mesh,
        out_type=jax.ShapeDtypeStruct((E, 128), jnp.float32),
        scratch_types=[pltpu.VMEM((_CH,), jnp.int32),
                       pltpu.VMEM((_CH,), jnp.int32),
                       pltpu.VMEM((_CH, 128), jnp.float32),
                       pltpu.VMEM((_CH, 128), jnp.float32),
                       pltpu.SemaphoreType.DMA, pltpu.SemaphoreType.DMA,
                       pltpu.SemaphoreType.DMA, pltpu.SemaphoreType.DMA,
                       pltpu.SemaphoreType.DMA, pltpu.SemaphoreType.DMA],
    )
    def gk(table_hbm, idx_hbm, out_hbm, i0, i1, r0, r1,
           si0, si1, sg0, sg1, sw0, sw1):
        wid = lax.axis_index("s") * 2 + lax.axis_index("c")
        ok_last = wid < _EXTRA
        ib = (i0, i1)
        rb = (r0, r1)
        si = (si0, si1)
        sg = (sg0, sg1)
        sw = (sw0, sw1)

        def base(j):
            return pl.multiple_of((wid + j * _NW) * _CH, _CH)

        def a_start(j):
            pltpu.async_copy(idx_hbm.at[pl.ds(base(j), _CH)], ib[j % 2],
                             si[j % 2])

        def a_wait(j):
            pltpu.make_async_copy(idx_hbm.at[pl.ds(base(j), _CH)], ib[j % 2],
                                  si[j % 2]).wait()

        def g_start(j):
            pltpu.async_copy(table_hbm.at[ib[j % 2]], rb[j % 2], sg[j % 2])

        def g_wait(j):
            pltpu.make_async_copy(table_hbm.at[ib[j % 2]], rb[j % 2],
                                  sg[j % 2]).wait()

        def w_start(j):
            pltpu.async_copy(rb[j % 2], out_hbm.at[pl.ds(base(j), _CH)],
                             sw[j % 2])

        def w_wait(j):
            pltpu.make_async_copy(rb[j % 2], out_hbm.at[pl.ds(base(j), _CH)],
                                  sw[j % 2]).wait()

        def maybe(j, fn):
            if j == _NSLOT - 1:
                pl.when(ok_last)(fn)
            else:
                fn()

        maybe(0, lambda: a_start(0))
        maybe(1, lambda: a_start(1))
        maybe(0, lambda: a_wait(0))
        maybe(0, lambda: g_start(0))
        for j in range(_NSLOT):
            if j + 1 < _NSLOT:
                maybe(j + 1, lambda j=j: a_wait(j + 1))
                if j >= 1:
                    maybe(j - 1, lambda j=j: w_wait(j - 1))
                maybe(j + 1, lambda j=j: g_start(j + 1))
            maybe(j, lambda j=j: g_wait(j))
            if j + 2 < _NSLOT:
                maybe(j + 2, lambda j=j: a_start(j + 2))
            maybe(j, lambda j=j: w_start(j))
        maybe(_NSLOT - 2, lambda: w_wait(_NSLOT - 2))
        maybe(_NSLOT - 1, lambda: w_wait(_NSLOT - 1))

    return gk(table, idx)


def _sc_scatter(out_e, dst, z128):
    mesh = plsc.VectorSubcoreMesh(core_axis_name="c", subcore_axis_name="s")

    @functools.partial(
        pl.kernel, mesh=mesh,
        out_type=jax.ShapeDtypeStruct((2, N, 128), jnp.float32),
        scratch_types=[pltpu.VMEM((_CH,), jnp.int32),
                       pltpu.VMEM((_CH,), jnp.int32),
                       pltpu.VMEM((_CH,), jnp.int32),
                       pltpu.VMEM((_CH, 128), jnp.float32),
                       pltpu.VMEM((_CH, 128), jnp.float32),
                       pltpu.VMEM((_CH, 128), jnp.float32),
                       pltpu.VMEM_SHARED((N, 128), jnp.float32),
                       pltpu.SemaphoreType.DMA, pltpu.SemaphoreType.DMA,
                       pltpu.SemaphoreType.DMA, pltpu.SemaphoreType.DMA,
                       pltpu.SemaphoreType.DMA, pltpu.SemaphoreType.DMA],
    )
    def sk(oute_hbm, dst_hbm, z_hbm, acc_hbm,
           i0, i1, i2, r0b, r1b, r2b, acc_sh,
           sa0, sa1, sa2, ss0, ss1, ss2):
        cid = lax.axis_index("c")
        sid = lax.axis_index("s")
        wid = sid * 2 + cid
        r0 = pl.multiple_of(sid * _RPT, 8)

        @pl.when(sid < 15)
        def _():
            pltpu.sync_copy(z_hbm.at[pl.ds(r0, _RPT)],
                            acc_sh.at[pl.ds(r0, _RPT)])

        @pl.when(sid == 15)
        def _():
            pltpu.sync_copy(z_hbm.at[pl.ds(15 * _RPT, _RPT_TAIL)],
                            acc_sh.at[pl.ds(15 * _RPT, _RPT_TAIL)])

        plsc.subcore_barrier()

        ok_last = wid < _EXTRA
        ib = (i0, i1, i2)
        rb = (r0b, r1b, r2b)
        sa = (sa0, sa1, sa2)
        ss = (ss0, ss1, ss2)

        def base(j):
            return pl.multiple_of((wid + j * _NW) * _CH, _CH)

        def load_start(j):
            pltpu.async_copy(dst_hbm.at[pl.ds(base(j), _CH)], ib[j % 3],
                             sa[j % 3])
            pltpu.async_copy(oute_hbm.at[pl.ds(base(j), _CH)], rb[j % 3],
                             sa[j % 3])

        def load_wait(j):
            pltpu.make_async_copy(dst_hbm.at[pl.ds(base(j), _CH)], ib[j % 3],
                                  sa[j % 3]).wait()
            pltpu.make_async_copy(oute_hbm.at[pl.ds(base(j), _CH)], rb[j % 3],
                                  sa[j % 3]).wait()

        def sc_start(j):
            pltpu.async_copy(rb[j % 3], acc_sh.at[ib[j % 3]], ss[j % 3],
                             add=True)

        def sc_wait(j):
            pltpu.make_async_copy(rb[j % 3], acc_sh.at[ib[j % 3]],
                                  ss[j % 3]).wait()

        def maybe(j, fn):
            if j == _NSLOT - 1:
                pl.when(ok_last)(fn)
            else:
                fn()

        maybe(0, lambda: load_start(0))
        maybe(1, lambda: load_start(1))
        for j in range(_NSLOT):
            if j + 2 < _NSLOT:
                if j >= 1:
                    maybe(j - 1, lambda j=j: sc_wait(j - 1))
                maybe(j + 2, lambda j=j: load_start(j + 2))
            maybe(j, lambda j=j: load_wait(j))
            maybe(j, lambda j=j: sc_start(j))
        for t in range(_NSLOT - 3, _NSLOT):
            maybe(t, lambda t=t: sc_wait(t))
        plsc.subcore_barrier()

        @pl.when(sid < 15)
        def _():
            pltpu.sync_copy(acc_sh.at[pl.ds(r0, _RPT)],
                            acc_hbm.at[cid, pl.ds(r0, _RPT)])

        @pl.when(sid == 15)
        def _():
            pltpu.sync_copy(acc_sh.at[pl.ds(15 * _RPT, _RPT_TAIL)],
                            acc_hbm.at[cid, pl.ds(15 * _RPT, _RPT_TAIL)])

    return sk(out_e, dst, z128)


def kernel(src_features, edge_sh, edge_emb, src, dst,
           mlp_w0, mlp_b0, mlp_w1, mlp_b1):
    table = jnp.pad(src_features, ((0, 0), (0, 64)))
    x1 = _sc_gather(table, src)
    out_e = _tc_main(x1, edge_sh, edge_emb, mlp_w0, mlp_b0, mlp_w1, mlp_b1)
    z128 = jnp.zeros((N, 128), jnp.float32)
    acc = _sc_scatter(out_e, dst, z128)
    return _tc_combine(acc)


# speedup vs baseline: 4.3811x; 1.0502x over previous
import functools

import numpy as np
import jax
import jax.numpy as jnp
from jax import lax
from jax.experimental import pallas as pl
from jax.experimental.pallas import tpu as pltpu
from jax.experimental.pallas import tpu_sc as plsc

E = 80000
N = 10000
MUL = 16

_PW_S = np.float32(1.0 / np.sqrt(32.0))
_PW_VI = np.float32(np.sqrt(3.0 / 32.0) / np.sqrt(3.0))
_INV3 = np.float32(1.0 / np.sqrt(3.0))
_ISQ2 = np.float32(1.0 / np.sqrt(2.0))

_IT = False


def _consts():
    r = np.kron(np.eye(16), np.ones((1, 16))).astype(jnp.bfloat16)
    s = np.kron(np.ones((16, 1)), np.eye(16)).astype(jnp.bfloat16)
    rv = np.kron(np.ones((1, 16)), np.eye(3)).astype(np.float32)
    t3r = np.zeros((48, 256), np.float32)
    for u in range(16):
        for k in range(3):
            for w in range(16):
                t3r[3 * u + k, 16 * u + w] = 1.0
    t3r = t3r.astype(jnp.bfloat16)
    ecr = np.zeros((48, 768), np.float32)
    for u in range(16):
        for j in range(3):
            for w in range(16):
                ecr[3 * u + j, 256 * j + 16 * u + w] = 1.0
    ecr = ecr.astype(jnp.bfloat16)
    sea = np.zeros((256, 48), np.float32)
    for u in range(16):
        for w in range(16):
            for k in range(3):
                sea[16 * u + w, 3 * w + k] = 1.0
    sea = sea.astype(jnp.bfloat16)
    seb = np.zeros((768, 48), np.float32)
    for k in range(3):
        for u in range(16):
            for w in range(16):
                seb[256 * k + 16 * u + w, 3 * w + k] = 1.0
    seb = seb.astype(jnp.bfloat16)
    return r, s, rv, t3r, ecr, sea, seb


def _tp_body(x1_ref, sh_ref, emb_ref, w0t_ref, b0_ref, w1t_ref, b1_ref,
             r_ref, s_ref, rv_ref, t3r_ref, ecr_ref, sea_ref, seb_ref,
             out_ref):
    bsz = out_ref.shape[0]
    h = emb_ref[...] @ w0t_ref[...] + b0_ref[...]
    h = 0.5 * h * (1.0 + lax.erf(h * _ISQ2))
    tpw = jnp.dot(h.astype(jnp.bfloat16), w1t_ref[...],
                  preferred_element_type=jnp.float32) + b1_ref[...]

    x1 = x1_ref[...]
    sh = sh_ref[...]
    s1 = x1[:, :16]
    x1v = x1[:, 16:64]
    s2 = sh[:, 0:1]
    v2 = sh[:, 1:4]

    bf = jnp.bfloat16
    v2rep = v2 @ rv_ref[...]
    tpwb = tpw.astype(bf)
    f32 = jnp.float32
    a011 = jnp.dot(s1.astype(bf), r_ref[...],
                   preferred_element_type=f32).astype(bf)
    a000 = a011 * s2.astype(bf)
    a110 = jnp.dot((x1v * (v2rep * _INV3)).astype(bf), t3r_ref[...],
                   preferred_element_type=f32).astype(bf)
    a101 = jnp.dot(x1v.astype(bf), ecr_ref[...],
                   preferred_element_type=f32).astype(bf)

    w000 = tpwb[:, 0:256]
    w011 = tpwb[:, 256:512]
    w101 = tpwb[:, 512:768]
    w110 = tpwb[:, 768:1024]
    w101x3 = jnp.concatenate([w101, w101, w101], axis=1)

    ps = a000 * w000 + a110 * w110
    out_s = _PW_S * jnp.dot(ps, s_ref[...],
                            preferred_element_type=jnp.float32)
    term1 = jnp.dot(a011 * w011, sea_ref[...],
                    preferred_element_type=jnp.float32) * v2rep
    term2 = jnp.dot(a101 * w101x3, seb_ref[...],
                    preferred_element_type=jnp.float32) * s2
    out_v = _PW_VI * (term1 + term2)
    pad = jnp.zeros((bsz, 63), jnp.float32)
    one = jnp.ones((bsz, 1), jnp.float32)
    out_ref[...] = jnp.concatenate([out_s, out_v, one, pad], axis=1)


def _tc_main(x1, edge_sh, edge_emb, w0, b0, w1, b1):
    bsz = 1600
    consts = [jnp.asarray(c) for c in _consts()]

    def dspec(cols):
        return pl.BlockSpec((bsz, cols), lambda i: (i, 0))

    def fspec(shape):
        return pl.BlockSpec(shape, lambda i: (0,) * len(shape))

    return pl.pallas_call(
        _tp_body,
        grid=(E // bsz,),
        in_specs=[dspec(128), dspec(4), dspec(64),
                  fspec((64, 64)), fspec((1, 64)),
                  fspec((64, 1024)), fspec((1, 1024)),
                  fspec((16, 256)), fspec((256, 16)), fspec((3, 48)),
                  fspec((48, 256)), fspec((48, 768)), fspec((256, 48)),
                  fspec((768, 48))],
        out_specs=dspec(128),
        out_shape=jax.ShapeDtypeStruct((E, 128), jnp.float32),
        interpret=_IT,
    )(x1, edge_sh, edge_emb, w0.T, b0.reshape(1, 64),
      w1.T.astype(jnp.bfloat16), b1.reshape(1, 1024), *consts)


def _combine_body(a0_ref, a1_ref, out_ref):
    a = a0_ref[...] + a1_ref[...]
    cnt = a[:, 64:65]
    out_ref[...] = a[:, :64] / jnp.maximum(cnt, 1.0)


def _tc_combine(acc):
    return pl.pallas_call(
        _combine_body,
        out_shape=jax.ShapeDtypeStruct((N, 64), jnp.float32),
        interpret=_IT,
    )(acc[0], acc[1])


_CH = 128
_NCH = E // _CH
_NW = 32
_BASE_CH = _NCH // _NW
_EXTRA = _NCH % _NW
_RPT = 632
_RPT_TAIL = N - 15 * _RPT


_NSLOT = _BASE_CH + 1


def _sc_gather(table, idx):
    mesh = plsc.VectorSubcoreMesh(core_axis_name="c", subcore_axis_name="s")

    @functools.partial(
        pl.kernel, mesh=mesh,
        out_type=jax.ShapeDtypeStruct((E, 128), jnp.float32),
        scratch_types=[pltpu.VMEM((_CH,), jnp.int32),
                       pltpu.VMEM((_CH,), jnp.int32),
                       pltpu.VMEM((_CH, 128), jnp.float32),
                       pltpu.VMEM((_CH, 128), jnp.float32),
                       pltpu.SemaphoreType.DMA, pltpu.SemaphoreType.DMA,
                       pltpu.SemaphoreType.DMA, pltpu.SemaphoreType.DMA,
                       pltpu.SemaphoreType.DMA, pltpu.SemaphoreType.DMA],
    )
    def gk(table_hbm, idx_hbm, out_hbm, i0, i1, r0, r1,
           si0, si1, sg0, sg1, sw0, sw1):
        wid = lax.axis_index("s") * 2 + lax.axis_index("c")
        ok_last = wid < _EXTRA
        ib = (i0, i1)
        rb = (r0, r1)
        si = (si0, si1)
        sg = (sg0, sg1)
        sw = (sw0, sw1)

        def base(j):
            return pl.multiple_of((wid + j * _NW) * _CH, _CH)

        def a_start(j):
            pltpu.async_copy(idx_hbm.at[pl.ds(base(j), _CH)], ib[j % 2],
                             si[j % 2])

        def a_wait(j):
            pltpu.make_async_copy(idx_hbm.at[pl.ds(base(j), _CH)], ib[j % 2],
                                  si[j % 2]).wait()

        def g_start(j):
            pltpu.async_copy(table_hbm.at[ib[j % 2]], rb[j % 2], sg[j % 2])

        def g_wait(j):
            pltpu.make_async_copy(table_hbm.at[ib[j % 2]], rb[j % 2],
                                  sg[j % 2]).wait()

        def w_start(j):
            pltpu.async_copy(rb[j % 2], out_hbm.at[pl.ds(base(j), _CH)],
                             sw[j % 2])

        def w_wait(j):
            pltpu.make_async_copy(rb[j % 2], out_hbm.at[pl.ds(base(j), _CH)],
                                  sw[j % 2]).wait()

        def maybe(j, fn):
            if j == _NSLOT - 1:
                pl.when(ok_last)(fn)
            else:
                fn()

        maybe(0, lambda: a_start(0))
        maybe(1, lambda: a_start(1))
        maybe(0, lambda: a_wait(0))
        maybe(0, lambda: g_start(0))
        for j in range(_NSLOT):
            if j + 1 < _NSLOT:
                maybe(j + 1, lambda j=j: a_wait(j + 1))
                if j >= 1:
                    maybe(j - 1, lambda j=j: w_wait(j - 1))
                maybe(j + 1, lambda j=j: g_start(j + 1))
            maybe(j, lambda j=j: g_wait(j))
            if j + 2 < _NSLOT:
                maybe(j + 2, lambda j=j: a_start(j + 2))
            maybe(j, lambda j=j: w_start(j))
        maybe(_NSLOT - 2, lambda: w_wait(_NSLOT - 2))
        maybe(_NSLOT - 1, lambda: w_wait(_NSLOT - 1))

    return gk(table, idx)


def _sc_scatter(out_e, dst, z128):
    mesh = plsc.VectorSubcoreMesh(core_axis_name="c", subcore_axis_name="s")

    @functools.partial(
        pl.kernel, mesh=mesh,
        out_type=jax.ShapeDtypeStruct((2, N, 128), jnp.float32),
        scratch_types=[pltpu.VMEM((_CH,), jnp.int32),
                       pltpu.VMEM((_CH,), jnp.int32),
                       pltpu.VMEM((_CH,), jnp.int32),
                       pltpu.VMEM((_CH, 128), jnp.float32),
                       pltpu.VMEM((_CH, 128), jnp.float32),
                       pltpu.VMEM((_CH, 128), jnp.float32),
                       pltpu.VMEM_SHARED((N, 128), jnp.float32),
                       pltpu.SemaphoreType.DMA, pltpu.SemaphoreType.DMA,
                       pltpu.SemaphoreType.DMA, pltpu.SemaphoreType.DMA,
                       pltpu.SemaphoreType.DMA, pltpu.SemaphoreType.DMA],
    )
    def sk(oute_hbm, dst_hbm, z_hbm, acc_hbm,
           i0, i1, i2, r0b, r1b, r2b, acc_sh,
           sa0, sa1, sa2, ss0, ss1, ss2):
        cid = lax.axis_index("c")
        sid = lax.axis_index("s")
        wid = sid * 2 + cid
        r0 = pl.multiple_of(sid * _RPT, 8)

        @pl.when(sid < 15)
        def _():
            pltpu.sync_copy(z_hbm.at[pl.ds(r0, _RPT)],
                            acc_sh.at[pl.ds(r0, _RPT)])

        @pl.when(sid == 15)
        def _():
            pltpu.sync_copy(z_hbm.at[pl.ds(15 * _RPT, _RPT_TAIL)],
                            acc_sh.at[pl.ds(15 * _RPT, _RPT_TAIL)])

        plsc.subcore_barrier()

        ok_last = wid < _EXTRA
        ib = (i0, i1, i2)
        rb = (r0b, r1b, r2b)
        sa = (sa0, sa1, sa2)
        ss = (ss0, ss1, ss2)

        def base(j):
            return pl.multiple_of((wid + j * _NW) * _CH, _CH)

        def load_start(j):
            pltpu.async_copy(dst_hbm.at[pl.ds(base(j), _CH)], ib[j % 3],
                             sa[j % 3])
            pltpu.async_copy(oute_hbm.at[pl.ds(base(j), _CH)], rb[j % 3],
                             sa[j % 3])

        def load_wait(j):
            pltpu.make_async_copy(dst_hbm.at[pl.ds(base(j), _CH)], ib[j % 3],
                                  sa[j % 3]).wait()
            pltpu.make_async_copy(oute_hbm.at[pl.ds(base(j), _CH)], rb[j % 3],
                                  sa[j % 3]).wait()

        def sc_start(j):
            pltpu.async_copy(rb[j % 3], acc_sh.at[ib[j % 3]], ss[j % 3],
                             add=True)

        def sc_wait(j):
            pltpu.make_async_copy(rb[j % 3], acc_sh.at[ib[j % 3]],
                                  ss[j % 3]).wait()

        def maybe(j, fn):
            if j == _NSLOT - 1:
                pl.when(ok_last)(fn)
            else:
                fn()

        maybe(0, lambda: load_start(0))
        maybe(1, lambda: load_start(1))
        for j in range(_NSLOT):
            if j + 2 < _NSLOT:
                if j >= 1:
                    maybe(j - 1, lambda j=j: sc_wait(j - 1))
                maybe(j + 2, lambda j=j: load_start(j + 2))
            maybe(j, lambda j=j: load_wait(j))
            maybe(j, lambda j=j: sc_start(j))
        for t in range(_NSLOT - 3, _NSLOT):
            maybe(t, lambda t=t: sc_wait(t))
        plsc.subcore_barrier()

        @pl.when(sid < 15)
        def _():
            pltpu.sync_copy(acc_sh.at[pl.ds(r0, _RPT)],
                            acc_hbm.at[cid, pl.ds(r0, _RPT)])

        @pl.when(sid == 15)
        def _():
            pltpu.sync_copy(acc_sh.at[pl.ds(15 * _RPT, _RPT_TAIL)],
                            acc_hbm.at[cid, pl.ds(15 * _RPT, _RPT_TAIL)])

    return sk(out_e, dst, z128)


def kernel(src_features, edge_sh, edge_emb, src, dst,
           mlp_w0, mlp_b0, mlp_w1, mlp_b1):
    table = jnp.pad(src_features, ((0, 0), (0, 64)))
    x1 = _sc_gather(table, src)
    out_e = _tc_main(x1, edge_sh, edge_emb, mlp_w0, mlp_b0, mlp_w1, mlp_b1)
    z128 = jnp.zeros((N, 128), jnp.float32)
    acc = _sc_scatter(out_e, dst, z128)
    return _tc_combine(acc)


# speedup vs baseline: 4.4723x; 1.0208x over previous
import functools

import numpy as np
import jax
import jax.numpy as jnp
from jax import lax
from jax.experimental import pallas as pl
from jax.experimental.pallas import tpu as pltpu
from jax.experimental.pallas import tpu_sc as plsc

E = 80000
N = 10000
MUL = 16

_PW_S = np.float32(1.0 / np.sqrt(32.0))
_PW_VI = np.float32(np.sqrt(3.0 / 32.0) / np.sqrt(3.0))
_INV3 = np.float32(1.0 / np.sqrt(3.0))
_ISQ2 = np.float32(1.0 / np.sqrt(2.0))

_IT = False


def _consts():
    r = np.kron(np.eye(16), np.ones((1, 16))).astype(jnp.bfloat16)
    s = np.kron(np.ones((16, 1)), np.eye(16)).astype(jnp.bfloat16)
    rv = np.kron(np.ones((1, 16)), np.eye(3)).astype(np.float32)
    t3r = np.zeros((48, 256), np.float32)
    for u in range(16):
        for k in range(3):
            for w in range(16):
                t3r[3 * u + k, 16 * u + w] = 1.0
    t3r = t3r.astype(jnp.bfloat16)
    ecr = np.zeros((48, 768), np.float32)
    for u in range(16):
        for j in range(3):
            for w in range(16):
                ecr[3 * u + j, 256 * j + 16 * u + w] = 1.0
    ecr = ecr.astype(jnp.bfloat16)
    sea = np.zeros((256, 48), np.float32)
    for u in range(16):
        for w in range(16):
            for k in range(3):
                sea[16 * u + w, 3 * w + k] = 1.0
    sea = sea.astype(jnp.bfloat16)
    seb = np.zeros((768, 48), np.float32)
    for k in range(3):
        for u in range(16):
            for w in range(16):
                seb[256 * k + 16 * u + w, 3 * w + k] = 1.0
    seb = seb.astype(jnp.bfloat16)
    return r, s, rv, t3r, ecr, sea, seb


def _tp_body(x1_ref, sh_ref, emb_ref, w0t_ref, b0_ref, w1t_ref, b1_ref,
             r_ref, s_ref, rv_ref, t3r_ref, ecr_ref, sea_ref, seb_ref,
             out_ref):
    bsz = out_ref.shape[0]
    h = emb_ref[...] @ w0t_ref[...] + b0_ref[...]
    h = 0.5 * h * (1.0 + lax.erf(h * _ISQ2))
    tpw = jnp.dot(h.astype(jnp.bfloat16), w1t_ref[...],
                  preferred_element_type=jnp.float32) + b1_ref[...]

    x1 = x1_ref[...]
    sh = sh_ref[...]
    s1 = x1[:, :16]
    x1v = x1[:, 16:64]
    s2 = sh[:, 0:1]
    v2 = sh[:, 1:4]

    bf = jnp.bfloat16
    v2rep = v2 @ rv_ref[...]
    tpwb = tpw.astype(bf)
    f32 = jnp.float32
    a011 = jnp.dot(s1.astype(bf), r_ref[...],
                   preferred_element_type=f32).astype(bf)
    a000 = a011 * s2.astype(bf)
    a110 = jnp.dot((x1v * (v2rep * _INV3)).astype(bf), t3r_ref[...],
                   preferred_element_type=f32).astype(bf)
    a101 = jnp.dot(x1v.astype(bf), ecr_ref[...],
                   preferred_element_type=f32).astype(bf)

    w000 = tpwb[:, 0:256]
    w011 = tpwb[:, 256:512]
    w101 = tpwb[:, 512:768]
    w110 = tpwb[:, 768:1024]
    w101x3 = jnp.concatenate([w101, w101, w101], axis=1)

    ps = a000 * w000 + a110 * w110
    out_s = _PW_S * jnp.dot(ps, s_ref[...],
                            preferred_element_type=jnp.float32)
    term1 = jnp.dot(a011 * w011, sea_ref[...],
                    preferred_element_type=jnp.float32) * v2rep
    term2 = jnp.dot(a101 * w101x3, seb_ref[...],
                    preferred_element_type=jnp.float32) * s2
    out_v = _PW_VI * (term1 + term2)
    pad = jnp.zeros((bsz, 63), jnp.float32)
    one = jnp.ones((bsz, 1), jnp.float32)
    out_ref[...] = jnp.concatenate([out_s, out_v, one, pad], axis=1)


def _tc_main(x1, edge_sh, edge_emb, w0, b0, w1, b1):
    bsz = 1600
    consts = [jnp.asarray(c) for c in _consts()]

    def dspec(cols):
        return pl.BlockSpec((bsz, cols), lambda i: (i, 0))

    def fspec(shape):
        return pl.BlockSpec(shape, lambda i: (0,) * len(shape))

    return pl.pallas_call(
        _tp_body,
        grid=(E // bsz,),
        in_specs=[dspec(128), dspec(4), dspec(64),
                  fspec((64, 64)), fspec((1, 64)),
                  fspec((64, 1024)), fspec((1, 1024)),
                  fspec((16, 256)), fspec((256, 16)), fspec((3, 48)),
                  fspec((48, 256)), fspec((48, 768)), fspec((256, 48)),
                  fspec((768, 48))],
        out_specs=dspec(128),
        out_shape=jax.ShapeDtypeStruct((E, 128), jnp.float32),
        interpret=_IT,
    )(x1, edge_sh, edge_emb, w0.T, b0.reshape(1, 64),
      w1.T.astype(jnp.bfloat16), b1.reshape(1, 1024), *consts)


def _combine_body(acc_ref, out_ref):
    a = acc_ref[0] + acc_ref[1]
    cnt = a[:, 64:65]
    out_ref[...] = a[:, :64] / jnp.maximum(cnt, 1.0)


def _tc_combine(acc):
    return pl.pallas_call(
        _combine_body,
        out_shape=jax.ShapeDtypeStruct((N, 64), jnp.float32),
        interpret=_IT,
    )(acc)


_CH = 128
_NCH = E // _CH
_NW = 32
_BASE_CH = _NCH // _NW
_EXTRA = _NCH % _NW
_RPT = 632
_RPT_TAIL = N - 15 * _RPT


_NSLOT = _BASE_CH + 1


def _sc_gather(table, idx):
    mesh = plsc.VectorSubcoreMesh(core_axis_name="c", subcore_axis_name="s")

    @functools.partial(
        pl.kernel, mesh=mesh,
        out_type=jax.ShapeDtypeStruct((E, 128), jnp.float32),
        scratch_types=[pltpu.VMEM((_CH,), jnp.int32),
                       pltpu.VMEM((_CH,), jnp.int32),
                       pltpu.VMEM((_CH, 128), jnp.float32),
                       pltpu.VMEM((_CH, 128), jnp.float32),
                       pltpu.SemaphoreType.DMA, pltpu.SemaphoreType.DMA,
                       pltpu.SemaphoreType.DMA, pltpu.SemaphoreType.DMA,
                       pltpu.SemaphoreType.DMA, pltpu.SemaphoreType.DMA],
    )
    def gk(table_hbm, idx_hbm, out_hbm, i0, i1, r0, r1,
           si0, si1, sg0, sg1, sw0, sw1):
        wid = lax.axis_index("s") * 2 + lax.axis_index("c")
        ok_last = wid < _EXTRA
        ib = (i0, i1)
        rb = (r0, r1)
        si = (si0, si1)
        sg = (sg0, sg1)
        sw = (sw0, sw1)

        def base(j):
            return pl.multiple_of((wid + j * _NW) * _CH, _CH)

        def a_start(j):
            pltpu.async_copy(idx_hbm.at[pl.ds(base(j), _CH)], ib[j % 2],
                             si[j % 2])

        def a_wait(j):
            pltpu.make_async_copy(idx_hbm.at[pl.ds(base(j), _CH)], ib[j % 2],
                                  si[j % 2]).wait()

        def g_start(j):
            pltpu.async_copy(table_hbm.at[ib[j % 2]], rb[j % 2], sg[j % 2])

        def g_wait(j):
            pltpu.make_async_copy(table_hbm.at[ib[j % 2]], rb[j % 2],
                                  sg[j % 2]).wait()

        def w_start(j):
            pltpu.async_copy(rb[j % 2], out_hbm.at[pl.ds(base(j), _CH)],
                             sw[j % 2])

        def w_wait(j):
            pltpu.make_async_copy(rb[j % 2], out_hbm.at[pl.ds(base(j), _CH)],
                                  sw[j % 2]).wait()

        def maybe(j, fn):
            if j == _NSLOT - 1:
                pl.when(ok_last)(fn)
            else:
                fn()

        maybe(0, lambda: a_start(0))
        maybe(1, lambda: a_start(1))
        maybe(0, lambda: a_wait(0))
        maybe(0, lambda: g_start(0))
        for j in range(_NSLOT):
            if j + 1 < _NSLOT:
                maybe(j + 1, lambda j=j: a_wait(j + 1))
                if j >= 1:
                    maybe(j - 1, lambda j=j: w_wait(j - 1))
                maybe(j + 1, lambda j=j: g_start(j + 1))
            maybe(j, lambda j=j: g_wait(j))
            if j + 2 < _NSLOT:
                maybe(j + 2, lambda j=j: a_start(j + 2))
            maybe(j, lambda j=j: w_start(j))
        maybe(_NSLOT - 2, lambda: w_wait(_NSLOT - 2))
        maybe(_NSLOT - 1, lambda: w_wait(_NSLOT - 1))

    return gk(table, idx)


def _sc_scatter(out_e, dst, z128):
    mesh = plsc.VectorSubcoreMesh(core_axis_name="c", subcore_axis_name="s")

    @functools.partial(
        pl.kernel, mesh=mesh,
        out_type=jax.ShapeDtypeStruct((2, N, 128), jnp.float32),
        scratch_types=[pltpu.VMEM((_CH,), jnp.int32),
                       pltpu.VMEM((_CH,), jnp.int32),
                       pltpu.VMEM((_CH,), jnp.int32),
                       pltpu.VMEM((_CH, 128), jnp.float32),
                       pltpu.VMEM((_CH, 128), jnp.float32),
                       pltpu.VMEM((_CH, 128), jnp.float32),
                       pltpu.VMEM_SHARED((N, 128), jnp.float32),
                       pltpu.SemaphoreType.DMA, pltpu.SemaphoreType.DMA,
                       pltpu.SemaphoreType.DMA, pltpu.SemaphoreType.DMA,
                       pltpu.SemaphoreType.DMA, pltpu.SemaphoreType.DMA],
    )
    def sk(oute_hbm, dst_hbm, z_hbm, acc_hbm,
           i0, i1, i2, r0b, r1b, r2b, acc_sh,
           sa0, sa1, sa2, ss0, ss1, ss2):
        cid = lax.axis_index("c")
        sid = lax.axis_index("s")
        wid = sid * 2 + cid
        r0 = pl.multiple_of(sid * _RPT, 8)

        @pl.when(sid < 15)
        def _():
            pltpu.sync_copy(z_hbm, acc_sh.at[pl.ds(r0, _RPT)])

        @pl.when(sid == 15)
        def _():
            pltpu.sync_copy(z_hbm.at[pl.ds(0, _RPT_TAIL)],
                            acc_sh.at[pl.ds(15 * _RPT, _RPT_TAIL)])

        plsc.subcore_barrier()

        ok_last = wid < _EXTRA
        ib = (i0, i1, i2)
        rb = (r0b, r1b, r2b)
        sa = (sa0, sa1, sa2)
        ss = (ss0, ss1, ss2)

        def base(j):
            return pl.multiple_of((wid + j * _NW) * _CH, _CH)

        def load_start(j):
            pltpu.async_copy(dst_hbm.at[pl.ds(base(j), _CH)], ib[j % 3],
                             sa[j % 3])
            pltpu.async_copy(oute_hbm.at[pl.ds(base(j), _CH)], rb[j % 3],
                             sa[j % 3])

        def load_wait(j):
            pltpu.make_async_copy(dst_hbm.at[pl.ds(base(j), _CH)], ib[j % 3],
                                  sa[j % 3]).wait()
            pltpu.make_async_copy(oute_hbm.at[pl.ds(base(j), _CH)], rb[j % 3],
                                  sa[j % 3]).wait()

        def sc_start(j):
            pltpu.async_copy(rb[j % 3], acc_sh.at[ib[j % 3]], ss[j % 3],
                             add=True)

        def sc_wait(j):
            pltpu.make_async_copy(rb[j % 3], acc_sh.at[ib[j % 3]],
                                  ss[j % 3]).wait()

        def maybe(j, fn):
            if j == _NSLOT - 1:
                pl.when(ok_last)(fn)
            else:
                fn()

        maybe(0, lambda: load_start(0))
        maybe(1, lambda: load_start(1))
        for j in range(_NSLOT):
            if j + 2 < _NSLOT:
                if j >= 1:
                    maybe(j - 1, lambda j=j: sc_wait(j - 1))
                maybe(j + 2, lambda j=j: load_start(j + 2))
            maybe(j, lambda j=j: load_wait(j))
            maybe(j, lambda j=j: sc_start(j))
        for t in range(_NSLOT - 3, _NSLOT):
            maybe(t, lambda t=t: sc_wait(t))
        plsc.subcore_barrier()

        @pl.when(sid < 15)
        def _():
            pltpu.sync_copy(acc_sh.at[pl.ds(r0, _RPT)],
                            acc_hbm.at[cid, pl.ds(r0, _RPT)])

        @pl.when(sid == 15)
        def _():
            pltpu.sync_copy(acc_sh.at[pl.ds(15 * _RPT, _RPT_TAIL)],
                            acc_hbm.at[cid, pl.ds(15 * _RPT, _RPT_TAIL)])

    return sk(out_e, dst, z128)


def kernel(src_features, edge_sh, edge_emb, src, dst,
           mlp_w0, mlp_b0, mlp_w1, mlp_b1):
    table = jnp.pad(src_features, ((0, 0), (0, 64)))
    x1 = _sc_gather(table, src)
    out_e = _tc_main(x1, edge_sh, edge_emb, mlp_w0, mlp_b0, mlp_w1, mlp_b1)
    z128 = jnp.zeros((_RPT, 128), jnp.float32)
    acc = _sc_scatter(out_e, dst, z128)
    return _tc_combine(acc)


# speedup vs baseline: 4.4746x; 1.0005x over previous
import functools

import numpy as np
import jax
import jax.numpy as jnp
from jax import lax
from jax.experimental import pallas as pl
from jax.experimental.pallas import tpu as pltpu
from jax.experimental.pallas import tpu_sc as plsc

E = 80000
N = 10000
MUL = 16

_PW_S = np.float32(1.0 / np.sqrt(32.0))
_PW_VI = np.float32(np.sqrt(3.0 / 32.0) / np.sqrt(3.0))
_INV3 = np.float32(1.0 / np.sqrt(3.0))
_ISQ2 = np.float32(1.0 / np.sqrt(2.0))

_IT = False


def _consts():
    r = np.kron(np.eye(16), np.ones((1, 16))).astype(jnp.bfloat16)
    s = np.kron(np.ones((16, 1)), np.eye(16)).astype(jnp.bfloat16)
    rv = np.kron(np.ones((1, 16)), np.eye(3)).astype(np.float32)
    t3r = np.zeros((48, 256), np.float32)
    for u in range(16):
        for k in range(3):
            for w in range(16):
                t3r[3 * u + k, 16 * u + w] = 1.0
    t3r = t3r.astype(jnp.bfloat16)
    ecr = np.zeros((48, 768), np.float32)
    for u in range(16):
        for j in range(3):
            for w in range(16):
                ecr[3 * u + j, 256 * j + 16 * u + w] = 1.0
    ecr = ecr.astype(jnp.bfloat16)
    sea = np.zeros((256, 48), np.float32)
    for u in range(16):
        for w in range(16):
            for k in range(3):
                sea[16 * u + w, 3 * w + k] = 1.0
    sea = sea.astype(jnp.bfloat16)
    seb = np.zeros((768, 48), np.float32)
    for k in range(3):
        for u in range(16):
            for w in range(16):
                seb[256 * k + 16 * u + w, 3 * w + k] = 1.0
    seb = seb.astype(jnp.bfloat16)
    return r, s, rv, t3r, ecr, sea, seb


def _tp_body(x1_ref, sh_ref, emb_ref, w0t_ref, b0_ref, w1t_ref, b1_ref,
             r_ref, s_ref, rv_ref, t3r_ref, ecr_ref, sea_ref, seb_ref,
             out_ref):
    bsz = out_ref.shape[0]
    h = emb_ref[...] @ w0t_ref[...] + b0_ref[...]
    h = 0.5 * h * (1.0 + lax.erf(h * _ISQ2))
    tpw = jnp.dot(h.astype(jnp.bfloat16), w1t_ref[...],
                  preferred_element_type=jnp.float32) + b1_ref[...]

    x1 = x1_ref[...]
    sh = sh_ref[...]
    s1 = x1[:, :16]
    x1v = x1[:, 16:64]
    s2 = sh[:, 0:1]
    v2 = sh[:, 1:4]

    bf = jnp.bfloat16
    v2rep = v2 @ rv_ref[...]
    tpwb = tpw.astype(bf)
    f32 = jnp.float32
    a011 = jnp.dot(s1.astype(bf), r_ref[...],
                   preferred_element_type=f32).astype(bf)
    a000 = a011 * s2.astype(bf)
    a110 = jnp.dot((x1v * (v2rep * _INV3)).astype(bf), t3r_ref[...],
                   preferred_element_type=f32).astype(bf)
    a101 = jnp.dot(x1v.astype(bf), ecr_ref[...],
                   preferred_element_type=f32).astype(bf)

    w000 = tpwb[:, 0:256]
    w011 = tpwb[:, 256:512]
    w101 = tpwb[:, 512:768]
    w110 = tpwb[:, 768:1024]
    w101x3 = jnp.concatenate([w101, w101, w101], axis=1)

    ps = a000 * w000 + a110 * w110
    out_s = _PW_S * jnp.dot(ps, s_ref[...],
                            preferred_element_type=jnp.float32)
    term1 = jnp.dot(a011 * w011, sea_ref[...],
                    preferred_element_type=jnp.float32) * v2rep
    term2 = jnp.dot(a101 * w101x3, seb_ref[...],
                    preferred_element_type=jnp.float32) * s2
    out_v = _PW_VI * (term1 + term2)
    pad = jnp.zeros((bsz, 63), jnp.float32)
    one = jnp.ones((bsz, 1), jnp.float32)
    out_ref[...] = jnp.concatenate([out_s, out_v, one, pad], axis=1)


def _tc_main(x1, edge_sh, edge_emb, w0, b0, w1, b1):
    bsz = 1600
    consts = [jnp.asarray(c) for c in _consts()]

    def dspec(cols):
        return pl.BlockSpec((bsz, cols), lambda i: (i, 0))

    def fspec(shape):
        return pl.BlockSpec(shape, lambda i: (0,) * len(shape))

    return pl.pallas_call(
        _tp_body,
        grid=(E // bsz,),
        in_specs=[dspec(128), dspec(4), dspec(64),
                  fspec((64, 64)), fspec((1, 64)),
                  fspec((64, 1024)), fspec((1, 1024)),
                  fspec((16, 256)), fspec((256, 16)), fspec((3, 48)),
                  fspec((48, 256)), fspec((48, 768)), fspec((256, 48)),
                  fspec((768, 48))],
        out_specs=dspec(128),
        out_shape=jax.ShapeDtypeStruct((E, 128), jnp.float32),
        interpret=_IT,
    )(x1, edge_sh, edge_emb, w0.T, b0.reshape(1, 64),
      w1.T.astype(jnp.bfloat16), b1.reshape(1, 1024), *consts)


def _combine_body(acc_ref, out_ref):
    a = acc_ref[0] + acc_ref[1]
    cnt = a[:, 64:65]
    out_ref[...] = a[:, :64] / jnp.maximum(cnt, 1.0)


def _tc_combine(acc):
    return pl.pallas_call(
        _combine_body,
        out_shape=jax.ShapeDtypeStruct((N, 64), jnp.float32),
        interpret=_IT,
    )(acc)


_CH = 128
_NCH = E // _CH
_NW = 32
_BASE_CH = _NCH // _NW
_EXTRA = _NCH % _NW
_RPT = 632
_RPT_TAIL = N - 15 * _RPT


_NSLOT = _BASE_CH + 1


def _sc_gather(table, idx):
    mesh = plsc.VectorSubcoreMesh(core_axis_name="c", subcore_axis_name="s")

    @functools.partial(
        pl.kernel, mesh=mesh,
        compiler_params=pltpu.CompilerParams(use_tc_tiling_on_sc=True),
        out_type=jax.ShapeDtypeStruct((E, 128), jnp.float32),
        scratch_types=[pltpu.VMEM((_CH,), jnp.int32),
                       pltpu.VMEM((_CH,), jnp.int32),
                       pltpu.VMEM((_CH, 128), jnp.float32),
                       pltpu.VMEM((_CH, 128), jnp.float32),
                       pltpu.SemaphoreType.DMA, pltpu.SemaphoreType.DMA,
                       pltpu.SemaphoreType.DMA, pltpu.SemaphoreType.DMA,
                       pltpu.SemaphoreType.DMA, pltpu.SemaphoreType.DMA],
    )
    def gk(table_hbm, idx_hbm, out_hbm, i0, i1, r0, r1,
           si0, si1, sg0, sg1, sw0, sw1):
        wid = lax.axis_index("s") * 2 + lax.axis_index("c")
        ok_last = wid < _EXTRA
        ib = (i0, i1)
        rb = (r0, r1)
        si = (si0, si1)
        sg = (sg0, sg1)
        sw = (sw0, sw1)

        def base(j):
            return pl.multiple_of((wid + j * _NW) * _CH, _CH)

        def a_start(j):
            pltpu.async_copy(idx_hbm.at[pl.ds(base(j), _CH)], ib[j % 2],
                             si[j % 2])

        def a_wait(j):
            pltpu.make_async_copy(idx_hbm.at[pl.ds(base(j), _CH)], ib[j % 2],
                                  si[j % 2]).wait()

        def g_start(j):
            pltpu.async_copy(table_hbm.at[ib[j % 2]], rb[j % 2], sg[j % 2])

        def g_wait(j):
            pltpu.make_async_copy(table_hbm.at[ib[j % 2]], rb[j % 2],
                                  sg[j % 2]).wait()

        def w_start(j):
            pltpu.async_copy(rb[j % 2], out_hbm.at[pl.ds(base(j), _CH)],
                             sw[j % 2])

        def w_wait(j):
            pltpu.make_async_copy(rb[j % 2], out_hbm.at[pl.ds(base(j), _CH)],
                                  sw[j % 2]).wait()

        def maybe(j, fn):
            if j == _NSLOT - 1:
                pl.when(ok_last)(fn)
            else:
                fn()

        maybe(0, lambda: a_start(0))
        maybe(1, lambda: a_start(1))
        maybe(0, lambda: a_wait(0))
        maybe(0, lambda: g_start(0))
        for j in range(_NSLOT):
            if j + 1 < _NSLOT:
                maybe(j + 1, lambda j=j: a_wait(j + 1))
                if j >= 1:
                    maybe(j - 1, lambda j=j: w_wait(j - 1))
                maybe(j + 1, lambda j=j: g_start(j + 1))
            maybe(j, lambda j=j: g_wait(j))
            if j + 2 < _NSLOT:
                maybe(j + 2, lambda j=j: a_start(j + 2))
            maybe(j, lambda j=j: w_start(j))
        maybe(_NSLOT - 2, lambda: w_wait(_NSLOT - 2))
        maybe(_NSLOT - 1, lambda: w_wait(_NSLOT - 1))

    return gk(table, idx)


def _sc_scatter(out_e, dst, z128):
    mesh = plsc.VectorSubcoreMesh(core_axis_name="c", subcore_axis_name="s")

    @functools.partial(
        pl.kernel, mesh=mesh,
        compiler_params=pltpu.CompilerParams(use_tc_tiling_on_sc=True),
        out_type=jax.ShapeDtypeStruct((2, N, 128), jnp.float32),
        scratch_types=[pltpu.VMEM((_CH,), jnp.int32),
                       pltpu.VMEM((_CH,), jnp.int32),
                       pltpu.VMEM((_CH,), jnp.int32),
                       pltpu.VMEM((_CH, 128), jnp.float32),
                       pltpu.VMEM((_CH, 128), jnp.float32),
                       pltpu.VMEM((_CH, 128), jnp.float32),
                       pltpu.VMEM_SHARED((N, 128), jnp.float32),
                       pltpu.SemaphoreType.DMA, pltpu.SemaphoreType.DMA,
                       pltpu.SemaphoreType.DMA, pltpu.SemaphoreType.DMA,
                       pltpu.SemaphoreType.DMA, pltpu.SemaphoreType.DMA],
    )
    def sk(oute_hbm, dst_hbm, z_hbm, acc_hbm,
           i0, i1, i2, r0b, r1b, r2b, acc_sh,
           sa0, sa1, sa2, ss0, ss1, ss2):
        cid = lax.axis_index("c")
        sid = lax.axis_index("s")
        wid = sid * 2 + cid
        r0 = pl.multiple_of(sid * _RPT, 8)

        @pl.when(sid < 15)
        def _():
            pltpu.sync_copy(z_hbm, acc_sh.at[pl.ds(r0, _RPT)])

        @pl.when(sid == 15)
        def _():
            pltpu.sync_copy(z_hbm.at[pl.ds(0, _RPT_TAIL)],
                            acc_sh.at[pl.ds(15 * _RPT, _RPT_TAIL)])

        plsc.subcore_barrier()

        ok_last = wid < _EXTRA
        ib = (i0, i1, i2)
        rb = (r0b, r1b, r2b)
        sa = (sa0, sa1, sa2)
        ss = (ss0, ss1, ss2)

        def base(j):
            return pl.multiple_of((wid + j * _NW) * _CH, _CH)

        def load_start(j):
            pltpu.async_copy(dst_hbm.at[pl.ds(base(j), _CH)], ib[j % 3],
                             sa[j % 3])
            pltpu.async_copy(oute_hbm.at[pl.ds(base(j), _CH)], rb[j % 3],
                             sa[j % 3])

        def load_wait(j):
            pltpu.make_async_copy(dst_hbm.at[pl.ds(base(j), _CH)], ib[j % 3],
                                  sa[j % 3]).wait()
            pltpu.make_async_copy(oute_hbm.at[pl.ds(base(j), _CH)], rb[j % 3],
                                  sa[j % 3]).wait()

        def sc_start(j):
            pltpu.async_copy(rb[j % 3], acc_sh.at[ib[j % 3]], ss[j % 3],
                             add=True)

        def sc_wait(j):
            pltpu.make_async_copy(rb[j % 3], acc_sh.at[ib[j % 3]],
                                  ss[j % 3]).wait()

        def maybe(j, fn):
            if j == _NSLOT - 1:
                pl.when(ok_last)(fn)
            else:
                fn()

        maybe(0, lambda: load_start(0))
        maybe(1, lambda: load_start(1))
        for j in range(_NSLOT):
            if j + 2 < _NSLOT:
                if j >= 1:
                    maybe(j - 1, lambda j=j: sc_wait(j - 1))
                maybe(j + 2, lambda j=j: load_start(j + 2))
            maybe(j, lambda j=j: load_wait(j))
            maybe(j, lambda j=j: sc_start(j))
        for t in range(_NSLOT - 3, _NSLOT):
            maybe(t, lambda t=t: sc_wait(t))
        plsc.subcore_barrier()

        @pl.when(sid < 15)
        def _():
            pltpu.sync_copy(acc_sh.at[pl.ds(r0, _RPT)],
                            acc_hbm.at[cid, pl.ds(r0, _RPT)])

        @pl.when(sid == 15)
        def _():
            pltpu.sync_copy(acc_sh.at[pl.ds(15 * _RPT, _RPT_TAIL)],
                            acc_hbm.at[cid, pl.ds(15 * _RPT, _RPT_TAIL)])

    return sk(out_e, dst, z128)


def kernel(src_features, edge_sh, edge_emb, src, dst,
           mlp_w0, mlp_b0, mlp_w1, mlp_b1):
    table = jnp.pad(src_features, ((0, 0), (0, 64)))
    x1 = _sc_gather(table, src)
    out_e = _tc_main(x1, edge_sh, edge_emb, mlp_w0, mlp_b0, mlp_w1, mlp_b1)
    z128 = jnp.zeros((_RPT, 128), jnp.float32)
    acc = _sc_scatter(out_e, dst, z128)
    return _tc_combine(acc)


# speedup vs baseline: 4.9994x; 1.1173x over previous
import functools

import numpy as np
import jax
import jax.numpy as jnp
from jax import lax
from jax.experimental import pallas as pl
from jax.experimental.pallas import tpu as pltpu
from jax.experimental.pallas import tpu_sc as plsc

E = 80000
N = 10000
MUL = 16

_PW_S = np.float32(1.0 / np.sqrt(32.0))
_PW_VI = np.float32(np.sqrt(3.0 / 32.0) / np.sqrt(3.0))
_INV3 = np.float32(1.0 / np.sqrt(3.0))
_ISQ2 = np.float32(1.0 / np.sqrt(2.0))

_IT = False


def _consts():
    r = np.kron(np.eye(16), np.ones((1, 16))).astype(jnp.bfloat16)
    s = np.kron(np.ones((16, 1)), np.eye(16)).astype(jnp.bfloat16)
    msh = np.zeros((4, 64), np.float32)
    for u in range(16):
        for k in range(3):
            msh[1 + k, 3 * u + k] = 1.0
    msh[0, 48] = 1.0
    t3r = np.zeros((48, 256), np.float32)
    for u in range(16):
        for k in range(3):
            for w in range(16):
                t3r[3 * u + k, 16 * u + w] = 1.0
    t3r = t3r.astype(jnp.bfloat16)
    ecr = np.zeros((48, 768), np.float32)
    for u in range(16):
        for j in range(3):
            for w in range(16):
                ecr[3 * u + j, 256 * j + 16 * u + w] = 1.0
    ecr = ecr.astype(jnp.bfloat16)
    sea = np.zeros((256, 48), np.float32)
    for u in range(16):
        for w in range(16):
            for k in range(3):
                sea[16 * u + w, 3 * w + k] = 1.0
    sea = sea.astype(jnp.bfloat16)
    seb = np.zeros((768, 48), np.float32)
    for k in range(3):
        for u in range(16):
            for w in range(16):
                seb[256 * k + 16 * u + w, 3 * w + k] = 1.0
    seb = seb.astype(jnp.bfloat16)
    return msh, r, s, t3r, ecr, sea, seb


def _tp_body(x1_ref, shx_ref, h0_ref, w1t_ref, b1_ref,
             r_ref, s_ref, t3r_ref, ecr_ref, sea_ref, seb_ref,
             out_ref):
    bsz = out_ref.shape[0]
    h0 = h0_ref[...]
    h = 0.5 * h0 * (1.0 + lax.erf(h0 * _ISQ2))
    tpw = jnp.dot(h.astype(jnp.bfloat16), w1t_ref[...],
                  preferred_element_type=jnp.float32) + b1_ref[...]

    x1 = x1_ref[...]
    shx = shx_ref[...]
    s1 = x1[:, :16]
    x1v = x1[:, 16:64]
    s2 = shx[:, 48:49]
    v2rep = shx[:, 0:48]

    bf = jnp.bfloat16
    tpwb = tpw.astype(bf)
    f32 = jnp.float32
    a011 = jnp.dot(s1.astype(bf), r_ref[...],
                   preferred_element_type=f32).astype(bf)
    a000 = a011 * s2.astype(bf)
    a110 = jnp.dot((x1v * (v2rep * _INV3)).astype(bf), t3r_ref[...],
                   preferred_element_type=f32).astype(bf)
    a101 = jnp.dot(x1v.astype(bf), ecr_ref[...],
                   preferred_element_type=f32).astype(bf)

    w000 = tpwb[:, 0:256]
    w011 = tpwb[:, 256:512]
    w101 = tpwb[:, 512:768]
    w110 = tpwb[:, 768:1024]
    w101x3 = jnp.concatenate([w101, w101, w101], axis=1)

    ps = a000 * w000 + a110 * w110
    out_s = _PW_S * jnp.dot(ps, s_ref[...],
                            preferred_element_type=jnp.float32)
    term1 = jnp.dot(a011 * w011, sea_ref[...],
                    preferred_element_type=jnp.float32) * v2rep
    term2 = jnp.dot(a101 * w101x3, seb_ref[...],
                    preferred_element_type=jnp.float32) * s2
    out_v = _PW_VI * (term1 + term2)
    pad = jnp.zeros((bsz, 63), jnp.float32)
    one = jnp.ones((bsz, 1), jnp.float32)
    out_ref[...] = jnp.concatenate([out_s, out_v, one, pad], axis=1)


def _tc_main(x1, edge_sh, edge_emb, w0, b0, w1, b1):
    bsz = 1600
    consts = [jnp.asarray(c) for c in _consts()]
    msh = consts[0]
    shx = edge_sh @ msh
    h0 = edge_emb @ w0.T + b0

    def dspec(cols):
        return pl.BlockSpec((bsz, cols), lambda i: (i, 0))

    def fspec(shape):
        return pl.BlockSpec(shape, lambda i: (0,) * len(shape))

    return pl.pallas_call(
        _tp_body,
        grid=(E // bsz,),
        in_specs=[dspec(128), dspec(64), dspec(64),
                  fspec((64, 1024)), fspec((1, 1024)),
                  fspec((16, 256)), fspec((256, 16)),
                  fspec((48, 256)), fspec((48, 768)), fspec((256, 48)),
                  fspec((768, 48))],
        out_specs=dspec(128),
        out_shape=jax.ShapeDtypeStruct((E, 128), jnp.float32),
        interpret=_IT,
    )(x1, shx, h0,
      w1.T.astype(jnp.bfloat16), b1.reshape(1, 1024), *consts[1:])


def _combine_body(acc_ref, out_ref):
    a = acc_ref[0] + acc_ref[1]
    cnt = a[:, 64:65]
    out_ref[...] = a[:, :64] / jnp.maximum(cnt, 1.0)


def _tc_combine(acc):
    return pl.pallas_call(
        _combine_body,
        out_shape=jax.ShapeDtypeStruct((N, 64), jnp.float32),
        interpret=_IT,
    )(acc)


_CH = 128
_NCH = E // _CH
_NW = 32
_BASE_CH = _NCH // _NW
_EXTRA = _NCH % _NW
_RPT = 632
_RPT_TAIL = N - 15 * _RPT


_NSLOT = _BASE_CH + 1


def _sc_gather(table, idx):
    mesh = plsc.VectorSubcoreMesh(core_axis_name="c", subcore_axis_name="s")

    @functools.partial(
        pl.kernel, mesh=mesh,
        compiler_params=pltpu.CompilerParams(use_tc_tiling_on_sc=True),
        out_type=jax.ShapeDtypeStruct((E, 128), jnp.float32),
        scratch_types=[pltpu.VMEM((_CH,), jnp.int32),
                       pltpu.VMEM((_CH,), jnp.int32),
                       pltpu.VMEM((_CH, 128), jnp.float32),
                       pltpu.VMEM((_CH, 128), jnp.float32),
                       pltpu.SemaphoreType.DMA, pltpu.SemaphoreType.DMA,
                       pltpu.SemaphoreType.DMA, pltpu.SemaphoreType.DMA,
                       pltpu.SemaphoreType.DMA, pltpu.SemaphoreType.DMA],
    )
    def gk(table_hbm, idx_hbm, out_hbm, i0, i1, r0, r1,
           si0, si1, sg0, sg1, sw0, sw1):
        wid = lax.axis_index("s") * 2 + lax.axis_index("c")
        ok_last = wid < _EXTRA
        ib = (i0, i1)
        rb = (r0, r1)
        si = (si0, si1)
        sg = (sg0, sg1)
        sw = (sw0, sw1)

        def base(j):
            return pl.multiple_of((wid + j * _NW) * _CH, _CH)

        def a_start(j):
            pltpu.async_copy(idx_hbm.at[pl.ds(base(j), _CH)], ib[j % 2],
                             si[j % 2])

        def a_wait(j):
            pltpu.make_async_copy(idx_hbm.at[pl.ds(base(j), _CH)], ib[j % 2],
                                  si[j % 2]).wait()

        def g_start(j):
            pltpu.async_copy(table_hbm.at[ib[j % 2]], rb[j % 2], sg[j % 2])

        def g_wait(j):
            pltpu.make_async_copy(table_hbm.at[ib[j % 2]], rb[j % 2],
                                  sg[j % 2]).wait()

        def w_start(j):
            pltpu.async_copy(rb[j % 2], out_hbm.at[pl.ds(base(j), _CH)],
                             sw[j % 2])

        def w_wait(j):
            pltpu.make_async_copy(rb[j % 2], out_hbm.at[pl.ds(base(j), _CH)],
                                  sw[j % 2]).wait()

        def maybe(j, fn):
            if j == _NSLOT - 1:
                pl.when(ok_last)(fn)
            else:
                fn()

        maybe(0, lambda: a_start(0))
        maybe(1, lambda: a_start(1))
        maybe(0, lambda: a_wait(0))
        maybe(0, lambda: g_start(0))
        for j in range(_NSLOT):
            if j + 1 < _NSLOT:
                maybe(j + 1, lambda j=j: a_wait(j + 1))
                if j >= 1:
                    maybe(j - 1, lambda j=j: w_wait(j - 1))
                maybe(j + 1, lambda j=j: g_start(j + 1))
            maybe(j, lambda j=j: g_wait(j))
            if j + 2 < _NSLOT:
                maybe(j + 2, lambda j=j: a_start(j + 2))
            maybe(j, lambda j=j: w_start(j))
        maybe(_NSLOT - 2, lambda: w_wait(_NSLOT - 2))
        maybe(_NSLOT - 1, lambda: w_wait(_NSLOT - 1))

    return gk(table, idx)


def _sc_scatter(out_e, dst, z128):
    mesh = plsc.VectorSubcoreMesh(core_axis_name="c", subcore_axis_name="s")

    @functools.partial(
        pl.kernel, mesh=mesh,
        compiler_params=pltpu.CompilerParams(use_tc_tiling_on_sc=True),
        out_type=jax.ShapeDtypeStruct((2, N, 128), jnp.float32),
        scratch_types=[pltpu.VMEM((_CH,), jnp.int32),
                       pltpu.VMEM((_CH,), jnp.int32),
                       pltpu.VMEM((_CH,), jnp.int32),
                       pltpu.VMEM((_CH, 128), jnp.float32),
                       pltpu.VMEM((_CH, 128), jnp.float32),
                       pltpu.VMEM((_CH, 128), jnp.float32),
                       pltpu.VMEM_SHARED((N, 128), jnp.float32),
                       pltpu.SemaphoreType.DMA, pltpu.SemaphoreType.DMA,
                       pltpu.SemaphoreType.DMA, pltpu.SemaphoreType.DMA,
                       pltpu.SemaphoreType.DMA, pltpu.SemaphoreType.DMA],
    )
    def sk(oute_hbm, dst_hbm, z_hbm, acc_hbm,
           i0, i1, i2, r0b, r1b, r2b, acc_sh,
           sa0, sa1, sa2, ss0, ss1, ss2):
        cid = lax.axis_index("c")
        sid = lax.axis_index("s")
        wid = sid * 2 + cid
        r0 = pl.multiple_of(sid * _RPT, 8)

        @pl.when(sid < 15)
        def _():
            pltpu.sync_copy(z_hbm, acc_sh.at[pl.ds(r0, _RPT)])

        @pl.when(sid == 15)
        def _():
            pltpu.sync_copy(z_hbm.at[pl.ds(0, _RPT_TAIL)],
                            acc_sh.at[pl.ds(15 * _RPT, _RPT_TAIL)])

        plsc.subcore_barrier()

        ok_last = wid < _EXTRA
        ib = (i0, i1, i2)
        rb = (r0b, r1b, r2b)
        sa = (sa0, sa1, sa2)
        ss = (ss0, ss1, ss2)

        def base(j):
            return pl.multiple_of((wid + j * _NW) * _CH, _CH)

        def load_start(j):
            pltpu.async_copy(dst_hbm.at[pl.ds(base(j), _CH)], ib[j % 3],
                             sa[j % 3])
            pltpu.async_copy(oute_hbm.at[pl.ds(base(j), _CH)], rb[j % 3],
                             sa[j % 3])

        def load_wait(j):
            pltpu.make_async_copy(dst_hbm.at[pl.ds(base(j), _CH)], ib[j % 3],
                                  sa[j % 3]).wait()
            pltpu.make_async_copy(oute_hbm.at[pl.ds(base(j), _CH)], rb[j % 3],
                                  sa[j % 3]).wait()

        def sc_start(j):
            pltpu.async_copy(rb[j % 3], acc_sh.at[ib[j % 3]], ss[j % 3],
                             add=True)

        def sc_wait(j):
            pltpu.make_async_copy(rb[j % 3], acc_sh.at[ib[j % 3]],
                                  ss[j % 3]).wait()

        def maybe(j, fn):
            if j == _NSLOT - 1:
                pl.when(ok_last)(fn)
            else:
                fn()

        maybe(0, lambda: load_start(0))
        maybe(1, lambda: load_start(1))
        for j in range(_NSLOT):
            if j + 2 < _NSLOT:
                if j >= 1:
                    maybe(j - 1, lambda j=j: sc_wait(j - 1))
                maybe(j + 2, lambda j=j: load_start(j + 2))
            maybe(j, lambda j=j: load_wait(j))
            maybe(j, lambda j=j: sc_start(j))
        for t in range(_NSLOT - 3, _NSLOT):
            maybe(t, lambda t=t: sc_wait(t))
        plsc.subcore_barrier()

        @pl.when(sid < 15)
        def _():
            pltpu.sync_copy(acc_sh.at[pl.ds(r0, _RPT)],
                            acc_hbm.at[cid, pl.ds(r0, _RPT)])

        @pl.when(sid == 15)
        def _():
            pltpu.sync_copy(acc_sh.at[pl.ds(15 * _RPT, _RPT_TAIL)],
                            acc_hbm.at[cid, pl.ds(15 * _RPT, _RPT_TAIL)])

    return sk(out_e, dst, z128)


def kernel(src_features, edge_sh, edge_emb, src, dst,
           mlp_w0, mlp_b0, mlp_w1, mlp_b1):
    table = jnp.pad(src_features, ((0, 0), (0, 64)))
    x1 = _sc_gather(table, src)
    out_e = _tc_main(x1, edge_sh, edge_emb, mlp_w0, mlp_b0, mlp_w1, mlp_b1)
    z128 = jnp.zeros((_RPT, 128), jnp.float32)
    acc = _sc_scatter(out_e, dst, z128)
    return _tc_combine(acc)
